# Initial kernel scaffold; baseline (speedup 1.0000x reference)
import jax
import jax.numpy as jnp
from jax.experimental import pallas as pl


def kernel(user_table, item_table, fusion_w, fusion_proj, edge_ubg, edge_view, edge_cart, edge_buy, edge_view_buy, edge_cart_buy):
    raise NotImplementedError("write your pallas kernel here")


# speedup vs baseline: 7.7383x; 7.7383x over previous
import functools

import jax
import jax.numpy as jnp
from jax import lax
from jax.experimental import pallas as pl
from jax.experimental.pallas import tpu as pltpu
from jax.experimental.pallas import tpu_sc as plsc

N_USERS = 25000
N_ITEMS = 25000
EMB = 128
E = 500000
N = (N_USERS + 1) + (N_ITEMS + 1)

NPAD = 50176
BLK = 512
GRID = NPAD // BLK

NCORES = 2
NTILES = 16
RPT = NPAD // NTILES
OROWS = RPT // 4
BW = 128
NB = 245
EP = NTILES * NB * BW
NCHUNK = 8
CW = EMB // NCHUNK
HW = 16
NDUMMY = NPAD - N

_MESH = plsc.VectorSubcoreMesh(
    core_axis_name="c", subcore_axis_name="s",
    num_cores=NCORES, num_subcores=NTILES)


def _zero_vmem(buf, ncols):
    zero16 = jnp.zeros((16,), jnp.float32)

    def body(i, _):
        for c0 in range(0, ncols, 16):
            buf[i, c0:c0 + 16] = zero16
        return 0

    lax.fori_loop(0, buf.shape[0], body, 0)


@functools.partial(
    pl.kernel,
    out_type=jax.ShapeDtypeStruct((2, 6, NPAD, HW), jnp.float32),
    mesh=_MESH,
    scratch_types=[
        pltpu.VMEM((NB, BW), jnp.int32),
        pltpu.VMEM((BW, HW), jnp.float32),
        pltpu.VMEM((OROWS, HW), jnp.float32),
        pltpu.VMEM_SHARED((NPAD, HW), jnp.float32),
    ],
    compiler_params=pltpu.CompilerParams(use_tc_tiling_on_sc=False),
)
def _hist_sc(edges_hbm, degs_hbm, ids, ones, zbuf, acc):
    c = lax.axis_index("c")
    s = lax.axis_index("s")
    row0 = s * RPT

    one16 = jnp.ones((16,), jnp.float32)

    def fill_ones(i, _):
        ones[i, :] = one16
        return 0

    lax.fori_loop(0, BW, fill_ones, 0)

    for b in range(6):
        pltpu.sync_copy(edges_hbm.at[b, c, s], ids)
        _zero_vmem(zbuf, HW)
        for z in range(RPT // OROWS):
            pltpu.sync_copy(zbuf, acc.at[pl.ds(row0 + z * OROWS, OROWS)])
        plsc.subcore_barrier()

        def body(j, _):
            pltpu.sync_copy(ones, acc.at[ids.at[j]], add=True)
            return 0

        lax.fori_loop(0, NB, body, 0)
        plsc.subcore_barrier()
        for z in range(RPT // OROWS):
            pltpu.sync_copy(acc.at[pl.ds(row0 + z * OROWS, OROWS)], zbuf)
            pltpu.sync_copy(zbuf, degs_hbm.at[c, b, pl.ds(row0 + z * OROWS, OROWS)])
        plsc.subcore_barrier()


@functools.partial(
    pl.kernel,
    out_type=jax.ShapeDtypeStruct((NCHUNK, NPAD, CW), jnp.float32),
    mesh=_MESH,
    scratch_types=[
        pltpu.VMEM((NB, BW), jnp.int32),
        pltpu.VMEM((NB, BW), jnp.int32),
        pltpu.VMEM((BW, CW), jnp.float32),
        pltpu.VMEM((BW, CW), jnp.float32),
        pltpu.VMEM((OROWS, CW), jnp.float32),
        pltpu.VMEM_SHARED((NPAD, CW), jnp.float32),
        pltpu.SemaphoreType.DMA,
        pltpu.SemaphoreType.DMA,
    ],
    compiler_params=pltpu.CompilerParams(use_tc_tiling_on_sc=False),
)
def _conv_sc(srcn_hbm, dst_hbm, xsflat_hbm, y_hbm,
             gidx, didx, rows0, rows1, zbuf, acc, sem0, sem1):
    c = lax.axis_index("c")
    s = lax.axis_index("s")
    row0 = s * RPT

    pltpu.sync_copy(dst_hbm.at[s], didx)

    for p in range(NCHUNK // NCORES):
        cc = c * (NCHUNK // NCORES) + p
        pltpu.sync_copy(srcn_hbm.at[cc, s], gidx)
        _zero_vmem(zbuf, CW)
        for z in range(RPT // OROWS):
            pltpu.sync_copy(zbuf, acc.at[pl.ds(row0 + z * OROWS, OROWS)])
        plsc.subcore_barrier()

        pltpu.async_copy(xsflat_hbm.at[gidx.at[0]], rows0, sem0)
        pltpu.async_copy(xsflat_hbm.at[gidx.at[1]], rows1, sem1)

        def body(i, _):
            j0 = 2 * i
            pltpu.make_async_copy(xsflat_hbm.at[gidx.at[j0]], rows0, sem0).wait()
            pltpu.sync_copy(rows0, acc.at[didx.at[j0]], add=True)

            @pl.when(j0 + 2 < NB)
            def _():
                pltpu.async_copy(xsflat_hbm.at[gidx.at[j0 + 2]], rows0, sem0)

            pltpu.make_async_copy(xsflat_hbm.at[gidx.at[j0 + 1]], rows1, sem1).wait()
            pltpu.sync_copy(rows1, acc.at[didx.at[j0 + 1]], add=True)

            @pl.when(j0 + 3 < NB)
            def _():
                pltpu.async_copy(xsflat_hbm.at[gidx.at[j0 + 3]], rows1, sem1)

            return 0

        lax.fori_loop(0, NB // 2, body, 0)
        pltpu.make_async_copy(xsflat_hbm.at[gidx.at[NB - 1]], rows0, sem0).wait()
        pltpu.sync_copy(rows0, acc.at[didx.at[NB - 1]], add=True)

        plsc.subcore_barrier()
        for z in range(RPT // OROWS):
            pltpu.sync_copy(acc.at[pl.ds(row0 + z * OROWS, OROWS)], zbuf)
            pltpu.sync_copy(zbuf, y_hbm.at[cc, pl.ds(row0 + z * OROWS, OROWS)])
        plsc.subcore_barrier()


def _pre_body(x_ref, deg_ref, o_ref):
    a = lax.rsqrt(jnp.maximum(deg_ref[:, 0:1], 1.0))
    o_ref[...] = x_ref[...] * a


_pre_tc = pl.pallas_call(
    _pre_body,
    grid=(GRID,),
    in_specs=[
        pl.BlockSpec((BLK, EMB), lambda i: (i, 0)),
        pl.BlockSpec((BLK, HW), lambda i: (i, 0)),
    ],
    out_specs=pl.BlockSpec((BLK, EMB), lambda i: (i, 0)),
    out_shape=jax.ShapeDtypeStruct((NPAD, EMB), jnp.float32),
)


def _post_body(y_ref, x_ref, deg_ref, o_ref):
    b = lax.rsqrt(jnp.maximum(deg_ref[:, 0:1], 1.0))
    t = y_ref[...] * b
    n = jnp.sqrt(jnp.sum(t * t, axis=1, keepdims=True))
    o_ref[...] = x_ref[...] + t / jnp.maximum(n, 1e-12)


_post_tc = pl.pallas_call(
    _post_body,
    grid=(GRID,),
    in_specs=[
        pl.BlockSpec((BLK, EMB), lambda i: (i, 0)),
        pl.BlockSpec((BLK, EMB), lambda i: (i, 0)),
        pl.BlockSpec((BLK, HW), lambda i: (i, 0)),
    ],
    out_specs=pl.BlockSpec((BLK, EMB), lambda i: (i, 0)),
    out_shape=jax.ShapeDtypeStruct((NPAD, EMB), jnp.float32),
)


def _fuse_body(w_ref, e0, e1, e2, e3, e4, e5, proj_ref, o_ref):
    acc = w_ref[0] * e0[...]
    for i, e in enumerate((e1, e2, e3, e4, e5)):
        acc = acc + w_ref[i + 1] * e[...]
    o_ref[...] = jnp.dot(acc, proj_ref[...],
                         preferred_element_type=jnp.float32)


_fuse_tc = pl.pallas_call(
    _fuse_body,
    grid=(GRID,),
    in_specs=[pl.BlockSpec(memory_space=pltpu.SMEM)]
    + [pl.BlockSpec((BLK, EMB), lambda i: (i, 0)) for _ in range(6)]
    + [pl.BlockSpec((EMB, EMB), lambda i: (0, 0))],
    out_specs=pl.BlockSpec((BLK, EMB), lambda i: (i, 0)),
    out_shape=jax.ShapeDtypeStruct((NPAD, EMB), jnp.float32),
)


def _conv_step(x, edge_pack, deg_out, deg_in):
    srcn, dstn = edge_pack
    xs = _pre_tc(x, deg_out)
    xs_flat = xs.reshape(NPAD * NCHUNK, CW)
    y = _conv_sc(srcn, dstn, xs_flat)
    y = jnp.concatenate([y[i] for i in range(NCHUNK)], axis=1)
    return _post_tc(y, x, deg_in)


def kernel(user_table, item_table, fusion_w, fusion_proj,
           edge_ubg, edge_view, edge_cart, edge_buy,
           edge_view_buy, edge_cart_buy):
    edges = [edge_ubg, edge_view, edge_cart, edge_buy,
             edge_view_buy, edge_cart_buy]
    edges = [e.astype(jnp.int32) for e in edges]

    x0 = jnp.concatenate([user_table, item_table], axis=0)
    x0 = jnp.pad(x0, ((0, NPAD - N), (0, 0)))

    pad_ids = (N + jnp.arange(EP - E, dtype=jnp.int32) % NDUMMY)[None, :]
    pad_ids = jnp.concatenate([pad_ids, pad_ids], axis=0)
    packs = []
    padded = []
    for e in edges:
        ep = jnp.concatenate([e, pad_ids], axis=1)
        padded.append(ep)
        srcn = (ep[0] * NCHUNK)[None, :] + jnp.arange(NCHUNK, dtype=jnp.int32)[:, None]
        packs.append((srcn.reshape(NCHUNK, NTILES, NB, BW),
                      ep[1].reshape(NTILES, NB, BW)))
    edges_all = jnp.stack([e.reshape(2, NTILES, NB, BW) for e in padded])

    degs = _hist_sc(edges_all)

    def _after(x, prev):
        return lax.optimization_barrier((x, prev))[0]

    emb_ubg = _conv_step(x0, packs[0], degs[0, 0], degs[1, 0])
    emb_view = _conv_step(emb_ubg, packs[1], degs[0, 1], degs[1, 1])
    emb_cart = _conv_step(_after(emb_ubg, emb_view), packs[2],
                          degs[0, 2], degs[1, 2])
    emb_buy = _conv_step(_after(emb_ubg, emb_cart), packs[3],
                         degs[0, 3], degs[1, 3])
    emb_vb = _conv_step(_after(emb_view, emb_buy), packs[4],
                        degs[0, 4], degs[1, 4])
    emb_cb = _conv_step(_after(emb_cart, emb_vb), packs[5],
                        degs[0, 5], degs[1, 5])

    w = jax.nn.softmax(fusion_w)
    fused = _fuse_tc(w, emb_ubg, emb_view, emb_cart, emb_buy,
                     emb_vb, emb_cb, fusion_proj)
    return fused[:N]
